# baseline (speedup 1.0000x reference)
import functools

import jax
import jax.numpy as jnp
from jax import lax
from jax.experimental import pallas as pl
from jax.experimental.pallas import tpu as pltpu
from jax.experimental.pallas import tpu_sc as plsc

_EPS = 1e-8
_TOPK = 8
_LANES = 16


def _prep_kernel(d_ref, dn_ref):
    d = d_ref[...]
    n = jnp.sqrt(jnp.sum(d * d, axis=1, keepdims=True))
    dn_ref[...] = (d / jnp.maximum(n, _EPS)).astype(jnp.bfloat16)


def _main_kernel(x_ref, dn_ref, idx_ref, w_ref):
    x = x_ref[...]
    xn = x / jnp.maximum(
        jnp.sqrt(jnp.sum(x * x, axis=1, keepdims=True)), _EPS)
    s = jax.lax.dot_general(
        xn.astype(jnp.bfloat16), dn_ref[...],
        (((1,), (1,)), ((), ())), preferred_element_type=jnp.float32)
    n = s.shape[1]
    iota = jax.lax.broadcasted_iota(jnp.int32, s.shape, 1).astype(jnp.float32)
    big = jnp.float32(n)
    v0 = None
    z = None
    cols = []
    wcols = []
    neg = jnp.float32(-jnp.inf)
    for k in range(_TOPK):
        m = jnp.max(s, axis=1, keepdims=True)
        idx = jnp.min(jnp.where(s == m, iota, big), axis=1, keepdims=True)
        if k == 0:
            v0 = m
            w = jnp.ones_like(m)
            z = w
        else:
            w = jnp.exp(m - v0)
            z = z + w
        wcols.append(w)
        s = jnp.where(iota == idx, neg, s)
        cols.append(idx)
    idx_ref[...] = jnp.concatenate(cols, axis=1).astype(jnp.int32)
    wn = jnp.concatenate(wcols, axis=1) / z
    w_ref[...] = jnp.broadcast_to(
        wn[:, :, None], (wn.shape[0], _TOPK, _LANES))


def _make_sc_gather(b, n, dd):
    info = plsc.get_sparse_core_info()
    nw = info.num_cores * info.num_subcores
    rw = b // nw
    ch = 16
    nch = rw // ch
    k = _TOPK
    mesh = plsc.VectorSubcoreMesh(core_axis_name="c", subcore_axis_name="s")

    @functools.partial(
        pl.kernel, mesh=mesh,
        out_type=jax.ShapeDtypeStruct((b, dd), jnp.float32),
        scratch_types=[
            pltpu.VMEM((ch * k,), jnp.int32),
            pltpu.VMEM((ch * k, dd), jnp.float32),
            pltpu.VMEM((ch * k, _LANES), jnp.float32),
            pltpu.VMEM((ch, dd), jnp.float32),
            pltpu.SemaphoreType.DMA,
        ],
    )
    def sc_gather(dict_hbm, idxf_hbm, w_hbm, out_hbm,
                  idx_v, rows_v, w_v, out_v, sem):
        wid = lax.axis_index("s") * info.num_cores + lax.axis_index("c")
        base = wid * rw

        def chunk(ci, carry):
            row0 = base + ci * ch
            pltpu.sync_copy(idxf_hbm.at[pl.ds(row0 * k, ch * k)], idx_v)
            pltpu.async_copy(dict_hbm.at[idx_v], rows_v, sem).wait()
            pltpu.sync_copy(w_hbm.at[pl.ds(row0 * k, ch * k)], w_v)
            for r in range(ch):
                for d in range(dd // _LANES):
                    acc = jnp.zeros((_LANES,), jnp.float32)
                    for kk in range(k):
                        j = r * k + kk
                        acc = acc + w_v[j] * rows_v[j, pl.ds(d * _LANES,
                                                             _LANES)]
                    out_v[r, pl.ds(d * _LANES, _LANES)] = acc
            pltpu.sync_copy(out_v, out_hbm.at[pl.ds(row0, ch)])
            return carry

        lax.fori_loop(0, nch, chunk, 0)

    return sc_gather


def kernel(inputs_flatten, dictionary):
    b, dd = inputs_flatten.shape
    n = dictionary.shape[0]
    blk_b = min(b, 256)
    norm_blk = min(n, 1024)

    dn = pl.pallas_call(
        _prep_kernel,
        grid=(n // norm_blk,),
        in_specs=[pl.BlockSpec((norm_blk, dd), lambda i: (i, 0))],
        out_specs=pl.BlockSpec((norm_blk, dd), lambda i: (i, 0)),
        out_shape=jax.ShapeDtypeStruct((n, dd), jnp.bfloat16),
    )(dictionary)

    idx, w3 = pl.pallas_call(
        _main_kernel,
        grid=(b // blk_b,),
        in_specs=[
            pl.BlockSpec((blk_b, dd), lambda i: (i, 0)),
            pl.BlockSpec((n, dd), lambda i: (0, 0)),
        ],
        out_specs=[
            pl.BlockSpec((blk_b, _TOPK), lambda i: (i, 0)),
            pl.BlockSpec((blk_b, _TOPK, _LANES), lambda i: (i, 0, 0)),
        ],
        out_shape=[
            jax.ShapeDtypeStruct((b, _TOPK), jnp.int32),
            jax.ShapeDtypeStruct((b, _TOPK, _LANES), jnp.float32),
        ],
    )(inputs_flatten, dn)

    idxf = idx.reshape(b * _TOPK)
    wf = w3.reshape(b * _TOPK, _LANES)
    emb = _make_sc_gather(b, n, dd)(dictionary, idxf, wf)
    return emb, idx


# speedup vs baseline: 1.4634x; 1.4634x over previous
import jax
import jax.numpy as jnp
from jax.experimental import pallas as pl

_EPS = 1e-8
_TOPK = 8


def _prep_kernel(d_ref, dn_ref, db_ref):
    d = d_ref[...]
    n = jnp.sqrt(jnp.sum(d * d, axis=1, keepdims=True))
    dn_ref[...] = (d / jnp.maximum(n, _EPS)).astype(jnp.bfloat16)
    db_ref[...] = d.astype(jnp.bfloat16)


def _main_kernel(x_ref, dn_ref, db_ref, emb_ref, idx_ref):
    x = x_ref[...]
    xn = x / jnp.maximum(
        jnp.sqrt(jnp.sum(x * x, axis=1, keepdims=True)), _EPS)
    s = jax.lax.dot_general(
        xn.astype(jnp.bfloat16), dn_ref[...],
        (((1,), (1,)), ((), ())), preferred_element_type=jnp.float32)
    n = s.shape[1]
    iota = jax.lax.broadcasted_iota(jnp.int32, s.shape, 1).astype(jnp.float32)
    big = jnp.float32(n)
    s0 = s
    v0 = None
    z = None
    cols = []
    neg = jnp.float32(-jnp.inf)
    for k in range(_TOPK):
        m = jnp.max(s, axis=1, keepdims=True)
        idx = jnp.min(jnp.where(s == m, iota, big), axis=1, keepdims=True)
        if k == 0:
            v0 = m
            z = jnp.ones_like(m)
        else:
            z = z + jnp.exp(m - v0)
        s = jnp.where(iota == idx, neg, s)
        cols.append(idx)
    idx_ref[...] = jnp.concatenate(cols, axis=1).astype(jnp.int32)
    u = jnp.where(s == neg, jnp.exp(s0 - v0), 0.0).astype(jnp.bfloat16)
    g = jax.lax.dot_general(
        u, db_ref[...],
        (((1,), (0,)), ((), ())), preferred_element_type=jnp.float32)
    emb_ref[...] = g / z


def kernel(inputs_flatten, dictionary):
    b, dd = inputs_flatten.shape
    n = dictionary.shape[0]
    blk_b = min(b, 256)
    norm_blk = min(n, 1024)

    dn, db = pl.pallas_call(
        _prep_kernel,
        grid=(n // norm_blk,),
        in_specs=[pl.BlockSpec((norm_blk, dd), lambda i: (i, 0))],
        out_specs=[
            pl.BlockSpec((norm_blk, dd), lambda i: (i, 0)),
            pl.BlockSpec((norm_blk, dd), lambda i: (i, 0)),
        ],
        out_shape=[
            jax.ShapeDtypeStruct((n, dd), jnp.bfloat16),
            jax.ShapeDtypeStruct((n, dd), jnp.bfloat16),
        ],
    )(dictionary)

    emb, idx = pl.pallas_call(
        _main_kernel,
        grid=(b // blk_b,),
        in_specs=[
            pl.BlockSpec((blk_b, dd), lambda i: (i, 0)),
            pl.BlockSpec((n, dd), lambda i: (0, 0)),
            pl.BlockSpec((n, dd), lambda i: (0, 0)),
        ],
        out_specs=[
            pl.BlockSpec((blk_b, dd), lambda i: (i, 0)),
            pl.BlockSpec((blk_b, _TOPK), lambda i: (i, 0)),
        ],
        out_shape=[
            jax.ShapeDtypeStruct((b, dd), jnp.float32),
            jax.ShapeDtypeStruct((b, _TOPK), jnp.int32),
        ],
    )(inputs_flatten, dn, db)
    return emb, idx
